# Initial kernel scaffold; baseline (speedup 1.0000x reference)
import jax
import jax.numpy as jnp
from jax.experimental import pallas as pl


def kernel(x, adj1, y, adj2, W1, b1, W2, b2, W3, b3, W4, b4, W5, b5, Wm, bm):
    raise NotImplementedError("write your pallas kernel here")


# speedup vs baseline: 1.8913x; 1.8913x over previous
import jax
import jax.numpy as jnp
from jax.experimental import pallas as pl
from jax.experimental.pallas import tpu as pltpu

_N = 4096
_NFEAT = 128
_F0, _F1, _F2 = 64, 68, 72
_SUMF = _F0 + _F1 + _F2
_H4 = _F0 * 2 + 4
_H5 = _F0 * 2
_W2C = _H5 + _H4
_FINC = _SUMF + _H4
_BM = 512


def _phase1_body(x_ref, adj_ref, w_ref, b_ref, out_ref, s_ref):
    i = pl.program_id(0)

    @pl.when(i == 0)
    def _():
        s = jnp.dot(x_ref[...].astype(jnp.bfloat16),
                    w_ref[...].astype(jnp.bfloat16),
                    preferred_element_type=jnp.float32)
        s_ref[...] = s.astype(jnp.bfloat16)

    z = jnp.dot(adj_ref[...].astype(jnp.bfloat16), s_ref[...],
                preferred_element_type=jnp.float32) + b_ref[...]
    sig = jax.nn.sigmoid(z)
    lane = jax.lax.broadcasted_iota(jnp.int32, sig.shape, 1)
    sec_mask = (lane >= _F0) & (lane < _F0 + _F1)
    msec = jnp.sum(jnp.where(sec_mask, sig, 0.0), axis=1,
                   keepdims=True) * (1.0 / _F1)
    out_ref[...] = jnp.where(lane < _F0 + _F1, sig, msec * sig)


def _phase1(x, adj, wc, bc):
    return pl.pallas_call(
        _phase1_body,
        grid=(_N // _BM,),
        in_specs=[
            pl.BlockSpec((_N, _NFEAT), lambda i: (0, 0)),
            pl.BlockSpec((_BM, _N), lambda i: (i, 0)),
            pl.BlockSpec((_NFEAT, _SUMF), lambda i: (0, 0)),
            pl.BlockSpec((1, _SUMF), lambda i: (0, 0)),
        ],
        out_specs=pl.BlockSpec((_BM, _SUMF), lambda i: (i, 0)),
        out_shape=jax.ShapeDtypeStruct((_N, _SUMF), jnp.float32),
        scratch_shapes=[pltpu.VMEM((_N, _SUMF), jnp.bfloat16)],
    )(x, adj, wc, bc)


def _phase2_body(lr_ref, adj_ref, w_ref, b_ref, wm_ref, bm_ref,
                 final_ref, fiv_ref, mlp_ref, s_ref):
    i = pl.program_id(0)

    @pl.when(i == 0)
    def _():
        s = jnp.dot(lr_ref[...].astype(jnp.bfloat16),
                    w_ref[...].astype(jnp.bfloat16),
                    preferred_element_type=jnp.float32)
        s_ref[...] = s.astype(jnp.bfloat16)

    z = jnp.dot(adj_ref[...].astype(jnp.bfloat16), s_ref[...],
                preferred_element_type=jnp.float32) + b_ref[...]
    fiv = z[:, :_H5]
    fou = z[:, _H5:]
    mlp = jnp.dot(fiv.astype(jnp.bfloat16), wm_ref[...].astype(jnp.bfloat16),
                  preferred_element_type=jnp.float32) + bm_ref[...]
    mlp = jnp.where(mlp >= 0.0, mlp, 0.01 * mlp)
    f3 = (mlp + fou) * 0.5
    row0 = pl.multiple_of(i * _BM, _BM)
    lr = lr_ref[pl.ds(row0, _BM), :]
    low = jnp.mean(lr, axis=1, keepdims=True) * lr + lr
    final_ref[...] = jnp.concatenate([low, f3], axis=1)
    fiv_ref[...] = fiv
    mlp_ref[...] = mlp


def _phase2(lr, adj, w45, b45, wmT, bm2):
    return pl.pallas_call(
        _phase2_body,
        grid=(_N // _BM,),
        in_specs=[
            pl.BlockSpec((_N, _SUMF), lambda i: (0, 0)),
            pl.BlockSpec((_BM, _N), lambda i: (i, 0)),
            pl.BlockSpec((_SUMF, _W2C), lambda i: (0, 0)),
            pl.BlockSpec((1, _W2C), lambda i: (0, 0)),
            pl.BlockSpec((_H5, _H4), lambda i: (0, 0)),
            pl.BlockSpec((1, _H4), lambda i: (0, 0)),
        ],
        out_specs=[
            pl.BlockSpec((_BM, _FINC), lambda i: (i, 0)),
            pl.BlockSpec((_BM, _H5), lambda i: (i, 0)),
            pl.BlockSpec((_BM, _H4), lambda i: (i, 0)),
        ],
        out_shape=[
            jax.ShapeDtypeStruct((_N, _FINC), jnp.float32),
            jax.ShapeDtypeStruct((_N, _H5), jnp.float32),
            jax.ShapeDtypeStruct((_N, _H4), jnp.float32),
        ],
        scratch_shapes=[pltpu.VMEM((_N, _W2C), jnp.bfloat16)],
    )(lr, adj, w45, b45, wmT, bm2)


def kernel(x, adj1, y, adj2, W1, b1, W2, b2, W3, b3, W4, b4, W5, b5, Wm, bm):
    wc = jnp.concatenate([W1, W2, W3], axis=1)
    bc = jnp.concatenate([b1, b2, b3]).reshape(1, _SUMF)
    w45 = jnp.concatenate([W5, W4], axis=1)
    b45 = jnp.concatenate([b5, b4]).reshape(1, _W2C)
    wmT = Wm.T
    bm2 = bm.reshape(1, _H4)

    x_lr = _phase1(x, adj1, wc, bc)
    y_lr = _phase1(y, adj2, wc, bc)
    x_final, x_fiv, x_mlp = _phase2(x_lr, adj1, w45, b45, wmT, bm2)
    y_final, y_fiv, y_mlp = _phase2(y_lr, adj2, w45, b45, wmT, bm2)
    return (x_lr, y_lr, x_final, y_final, x_fiv, x_mlp, y_fiv, y_mlp)


# speedup vs baseline: 1.9914x; 1.0529x over previous
import jax
import jax.numpy as jnp
from jax.experimental import pallas as pl
from jax.experimental.pallas import tpu as pltpu

_N = 4096
_NFEAT = 128
_F0, _F1, _F2 = 64, 68, 72
_SUMF = _F0 + _F1 + _F2
_H4 = _F0 * 2 + 4
_H5 = _F0 * 2
_W2C = _H5 + _H4
_FINC = _SUMF + _H4
_BM = 512
_NB = _N // _BM


def _branch_body(x_ref, adj_ref, wc_ref, bc_ref, w45_ref, b45_ref, wm_ref,
                 bm_ref, lr_out, final_out, fiv_out, mlp_out,
                 s1_ref, s2_ref, lr_ref):
    p = pl.program_id(0)
    i = pl.program_id(1)
    row0 = pl.multiple_of(i * _BM, _BM)

    @pl.when((p == 0) & (i == 0))
    def _():
        s = jnp.dot(x_ref[...].astype(jnp.bfloat16),
                    wc_ref[...].astype(jnp.bfloat16),
                    preferred_element_type=jnp.float32)
        s1_ref[...] = s.astype(jnp.bfloat16)

    @pl.when(p == 0)
    def _():
        z = jnp.dot(adj_ref[...].astype(jnp.bfloat16), s1_ref[...],
                    preferred_element_type=jnp.float32) + bc_ref[...]
        sig = jax.nn.sigmoid(z)
        lane = jax.lax.broadcasted_iota(jnp.int32, sig.shape, 1)
        sec_mask = (lane >= _F0) & (lane < _F0 + _F1)
        msec = jnp.sum(jnp.where(sec_mask, sig, 0.0), axis=1,
                       keepdims=True) * (1.0 / _F1)
        lr_blk = jnp.where(lane < _F0 + _F1, sig, msec * sig)
        lr_ref[pl.ds(row0, _BM), :] = lr_blk
        lr_out[...] = lr_blk

    @pl.when((p == 1) & (i == 0))
    def _():
        s = jnp.dot(lr_ref[...].astype(jnp.bfloat16),
                    w45_ref[...].astype(jnp.bfloat16),
                    preferred_element_type=jnp.float32)
        s2_ref[...] = s.astype(jnp.bfloat16)

    @pl.when(p == 1)
    def _():
        z = jnp.dot(adj_ref[...].astype(jnp.bfloat16), s2_ref[...],
                    preferred_element_type=jnp.float32) + b45_ref[...]
        fiv = z[:, :_H5]
        fou = z[:, _H5:]
        mlp = jnp.dot(fiv.astype(jnp.bfloat16),
                      wm_ref[...].astype(jnp.bfloat16),
                      preferred_element_type=jnp.float32) + bm_ref[...]
        mlp = jnp.where(mlp >= 0.0, mlp, 0.01 * mlp)
        f3 = (mlp + fou) * 0.5
        lr = lr_ref[pl.ds(row0, _BM), :]
        low = jnp.mean(lr, axis=1, keepdims=True) * lr + lr
        final_out[...] = jnp.concatenate([low, f3], axis=1)
        fiv_out[...] = fiv
        mlp_out[...] = mlp


def _branch(x, adj, wc, bc, w45, b45, wmT, bm2):
    return pl.pallas_call(
        _branch_body,
        grid=(2, _NB),
        in_specs=[
            pl.BlockSpec((_N, _NFEAT), lambda p, i: (0, 0)),
            pl.BlockSpec((_BM, _N), lambda p, i: (i, 0)),
            pl.BlockSpec((_NFEAT, _SUMF), lambda p, i: (0, 0)),
            pl.BlockSpec((1, _SUMF), lambda p, i: (0, 0)),
            pl.BlockSpec((_SUMF, _W2C), lambda p, i: (0, 0)),
            pl.BlockSpec((1, _W2C), lambda p, i: (0, 0)),
            pl.BlockSpec((_H5, _H4), lambda p, i: (0, 0)),
            pl.BlockSpec((1, _H4), lambda p, i: (0, 0)),
        ],
        out_specs=[
            pl.BlockSpec((_BM, _SUMF), lambda p, i: (jnp.where(p == 0, i, _NB - 1), 0)),
            pl.BlockSpec((_BM, _FINC), lambda p, i: (jnp.where(p == 0, 0, i), 0)),
            pl.BlockSpec((_BM, _H5), lambda p, i: (jnp.where(p == 0, 0, i), 0)),
            pl.BlockSpec((_BM, _H4), lambda p, i: (jnp.where(p == 0, 0, i), 0)),
        ],
        out_shape=[
            jax.ShapeDtypeStruct((_N, _SUMF), jnp.float32),
            jax.ShapeDtypeStruct((_N, _FINC), jnp.float32),
            jax.ShapeDtypeStruct((_N, _H5), jnp.float32),
            jax.ShapeDtypeStruct((_N, _H4), jnp.float32),
        ],
        scratch_shapes=[
            pltpu.VMEM((_N, _SUMF), jnp.bfloat16),
            pltpu.VMEM((_N, _W2C), jnp.bfloat16),
            pltpu.VMEM((_N, _SUMF), jnp.float32),
        ],
    )(x, adj, wc, bc, w45, b45, wmT, bm2)


def kernel(x, adj1, y, adj2, W1, b1, W2, b2, W3, b3, W4, b4, W5, b5, Wm, bm):
    wc = jnp.concatenate([W1, W2, W3], axis=1)
    bc = jnp.concatenate([b1, b2, b3]).reshape(1, _SUMF)
    w45 = jnp.concatenate([W5, W4], axis=1)
    b45 = jnp.concatenate([b5, b4]).reshape(1, _W2C)
    wmT = Wm.T
    bm2 = bm.reshape(1, _H4)

    x_lr, x_final, x_fiv, x_mlp = _branch(x, adj1, wc, bc, w45, b45, wmT, bm2)
    y_lr, y_final, y_fiv, y_mlp = _branch(y, adj2, wc, bc, w45, b45, wmT, bm2)
    return (x_lr, y_lr, x_final, y_final, x_fiv, x_mlp, y_fiv, y_mlp)
